# baseline (speedup 1.0000x reference)
import jax
import jax.numpy as jnp
from jax.experimental import pallas as pl
from jax.experimental.pallas import tpu as pltpu

N = 4096
F = 16
BM = 512
NBLK = N // BM
NBUF = 3


def _fused_kernel(adj_hbm, w_hbm, out_hbm,
                  abuf, obuf, wbuf, zb, zt,
                  rsem, wsem, psem):
    pltpu.make_async_copy(w_hbm, wbuf, psem).start()

    for k in range(min(NBUF - 1, NBLK)):
        pltpu.make_async_copy(
            adj_hbm.at[pl.ds(k * BM, BM), :], abuf.at[k % NBUF],
            rsem.at[k % NBUF]).start()

    pltpu.make_async_copy(w_hbm, wbuf, psem).wait()
    w_bf16 = wbuf[...].astype(jnp.bfloat16)

    for k in range(NBLK):
        s = k % NBUF
        pltpu.make_async_copy(
            adj_hbm.at[pl.ds(k * BM, BM), :], abuf.at[s], rsem.at[s]).wait()
        if k + NBUF - 1 < NBLK:
            pltpu.make_async_copy(
                adj_hbm.at[pl.ds((k + NBUF - 1) * BM, BM), :],
                abuf.at[(k + NBUF - 1) % NBUF],
                rsem.at[(k + NBUF - 1) % NBUF]).start()
        zi = jnp.dot(abuf[s].astype(jnp.bfloat16), w_bf16,
                     preferred_element_type=jnp.float32)
        zi16 = zi.astype(jnp.bfloat16)
        zb[pl.ds(k * BM, BM), :] = zi16
        zt[:, pl.ds(k * BM, BM)] = zi16.T

    ztv = zt[...]
    for j in range(NBLK):
        s = j % NBUF
        if j >= NBUF:
            pltpu.make_async_copy(
                obuf.at[s], out_hbm.at[pl.ds((j - NBUF) * BM, BM), :],
                wsem.at[s]).wait()
        obuf[s] = jnp.dot(zb[pl.ds(j * BM, BM), :], ztv,
                          preferred_element_type=jnp.float32)
        pltpu.make_async_copy(
            obuf.at[s], out_hbm.at[pl.ds(j * BM, BM), :], wsem.at[s]).start()

    for j in range(max(NBLK - NBUF, 0), NBLK):
        s = j % NBUF
        pltpu.make_async_copy(
            obuf.at[s], out_hbm.at[pl.ds(j * BM, BM), :], wsem.at[s]).wait()


@jax.jit
def kernel(adj, W):
    return pl.pallas_call(
        _fused_kernel,
        in_specs=[
            pl.BlockSpec(memory_space=pltpu.MemorySpace.HBM),
            pl.BlockSpec(memory_space=pltpu.MemorySpace.HBM),
        ],
        out_specs=pl.BlockSpec(memory_space=pltpu.MemorySpace.HBM),
        out_shape=jax.ShapeDtypeStruct((N, N), jnp.float32),
        scratch_shapes=[
            pltpu.VMEM((NBUF, BM, N), jnp.float32),
            pltpu.VMEM((NBUF, BM, N), jnp.float32),
            pltpu.VMEM((N, F), jnp.float32),
            pltpu.VMEM((N, F), jnp.bfloat16),
            pltpu.VMEM((F, N), jnp.bfloat16),
            pltpu.SemaphoreType.DMA((NBUF,)),
            pltpu.SemaphoreType.DMA((NBUF,)),
            pltpu.SemaphoreType.DMA,
        ],
    )(adj, W)
